# baseline (speedup 1.0000x reference)
import functools

import jax
import jax.numpy as jnp
from jax import lax
from jax.experimental import pallas as pl
from jax.experimental.pallas import tpu as pltpu
from jax.experimental.pallas import tpu_sc as plsc

_NC = 2
_NS = 16
_NW = _NC * _NS

_V = 1000
_D = 1000
_DP = 1024
_B = 1024
_S = 50
_SP = 56

_BPW = _B // _NW


def _gather_body(idx_hbm, table_hbm, out_hbm, idx_v, buf0, buf1,
                 gsem, ssem0, ssem1):
    wid = lax.axis_index("s") * _NC + lax.axis_index("c")
    base = wid * _BPW

    pltpu.sync_copy(idx_hbm.at[wid], idx_v)

    bufs = (buf0, buf1)
    ssems = (ssem0, ssem1)

    def body(i, carry):
        for b in range(2):
            k = 2 * i + b

            @pl.when(i >= 1)
            def _wait_prev():
                pltpu.make_async_copy(
                    bufs[b], out_hbm.at[base + k - 2], ssems[b]
                ).wait()

            pltpu.async_copy(
                table_hbm.at[idx_v.at[k]], bufs[b], gsem
            ).wait()

            pltpu.async_copy(bufs[b], out_hbm.at[base + k], ssems[b])
        return carry

    lax.fori_loop(0, _BPW // 2, body, None)

    pltpu.make_async_copy(buf0, out_hbm.at[base + _BPW - 2], ssem0).wait()
    pltpu.make_async_copy(buf1, out_hbm.at[base + _BPW - 1], ssem1).wait()


_mesh = plsc.VectorSubcoreMesh(
    core_axis_name="c", subcore_axis_name="s",
    num_cores=_NC, num_subcores=_NS,
)

_gather_call = functools.partial(
    pl.kernel,
    out_type=jax.ShapeDtypeStruct((_B, _SP, _DP), jnp.float32),
    mesh=_mesh,
    compiler_params=pltpu.CompilerParams(use_tc_tiling_on_sc=True),
    scratch_types=[
        pltpu.VMEM((_BPW, _SP), jnp.int32),
        pltpu.VMEM((_SP, _DP), jnp.float32),
        pltpu.VMEM((_SP, _DP), jnp.float32),
        pltpu.SemaphoreType.DMA,
        pltpu.SemaphoreType.DMA,
        pltpu.SemaphoreType.DMA,
    ],
)(_gather_body)


@jax.jit
def kernel(idx, table):
    idxp = jnp.pad(idx.astype(jnp.int32), ((0, 0), (0, _SP - _S)))
    idx3 = idxp.reshape(_NW, _BPW, _SP)
    table_p = jnp.pad(table, ((0, 0), (0, _DP - _D)))
    out_p = _gather_call(idx3, table_p)
    return out_p[:, :_S, :_D]
